# baseline (speedup 1.0000x reference)
import functools
import math

import jax
import jax.numpy as jnp
from jax import lax
from jax.experimental import pallas as pl
from jax.experimental.pallas import tpu as pltpu
from jax.experimental.pallas import tpu_sc as plsc

_LN2 = 0.6931471805599453
_SQRT2 = 1.4142135623730951
R_SC = 256
_CH = 10000
_BR = 16
_BC = 4096


_LNC = (1.0000009643975831, -0.5000114503774584, 0.33314673808549666,
        -0.24908289184720672, 0.20491759650007188, -0.18680751427133602,
        0.11931054435732005)


def _fast_ln(x):
    i = lax.bitcast_convert_type(x, jnp.int32)
    e = (i >> 23) - 127
    m = lax.bitcast_convert_type(
        (i & 0x007FFFFF) | 0x3F800000, jnp.float32)
    big = m >= _SQRT2
    m = jnp.where(big, m * 0.5, m)
    ef = e.astype(jnp.float32) + jnp.where(big, 1.0, 0.0)
    t = m - 1.0
    q = _LNC[6]
    for c in _LNC[5::-1]:
        q = q * t + c
    return t * q + ef * _LN2


def _sc_rowsum(pred, target, n_rows):
    ncols = pred.shape[1]
    info = plsc.get_sparse_core_info()
    nw = info.num_cores * info.num_subcores
    gw = n_rows // (8 * nw)
    cw = 2048
    nch = ncols // cw
    tail = ncols - nch * cw
    mesh = plsc.VectorSubcoreMesh(core_axis_name="c", subcore_axis_name="s")

    @functools.partial(
        pl.kernel,
        out_type=jax.ShapeDtypeStruct((n_rows * 16,), jnp.float32),
        mesh=mesh,
        scratch_types=[
            pltpu.VMEM((8, cw), jnp.float32),
            pltpu.VMEM((8, cw), jnp.float32),
            pltpu.VMEM((8, cw), jnp.float32),
            pltpu.VMEM((8, tail), jnp.float32),
            pltpu.VMEM((8,), jnp.int32),
            pltpu.VMEM((128,), jnp.float32),
            pltpu.SemaphoreType.DMA,
            pltpu.SemaphoreType.DMA,
            pltpu.SemaphoreType.DMA,
            pltpu.SemaphoreType.DMA,
        ],
        compiler_params=pltpu.CompilerParams(needs_layout_passes=False),
    )
    def rk(pred_hbm, tgt_hbm, out_hbm, buf0, buf1, buf2, tbuf, trow, o16,
           sem0, sem1, sem2, tsem):
        wid = lax.axis_index("s") * info.num_cores + lax.axis_index("c")
        iota16 = lax.broadcasted_iota(jnp.int32, (16,), 0)
        bufs = (buf0, buf1, buf2)
        sems = (sem0, sem1, sem2)

        for g in range(gw):
            row0 = (wid * gw + g) * 8
            pltpu.sync_copy(tgt_hbm.at[pl.ds(row0, 8)], trow)
            tvecs = [
                plsc.load_gather(trow, [jnp.full((16,), r, jnp.int32)])
                for r in range(8)
            ]
            pltpu.async_copy(
                pred_hbm.at[pl.ds(row0, 8), pl.ds(nch * cw, tail)],
                tbuf, tsem).start()

            def rowsums(buf, col0, width, accs):
                out = []
                for r in range(8):
                    def body(i, a, r=r, buf=buf, col0=col0):
                        for u in range(4):
                            jj = i * 4 + u
                            cols = col0 + jj * 16 + iota16
                            v = _fast_ln(buf[r, pl.ds(jj * 16, 16)])
                            a = a + jnp.where(cols == tvecs[r], 0.0, v)
                        return a
                    out.append(lax.fori_loop(0, width // 64, body, accs[r]))
                return tuple(out)

            def issue(k, slot):
                pltpu.make_async_copy(
                    pred_hbm.at[pl.ds(row0, 8), pl.ds(k * cw, cw)],
                    bufs[slot], sems[slot]).start()

            def wait(slot):
                pltpu.make_async_copy(
                    pred_hbm.at[pl.ds(row0, 8), pl.ds(0, cw)],
                    bufs[slot], sems[slot]).wait()

            issue(0, 0)
            issue(1, 1)
            zero = jnp.zeros((16,), jnp.float32)
            accs = (zero,) * 8

            def triple(q, accs):
                c = q * 3
                wait(0)
                accs = rowsums(buf0, c * cw, cw, accs)

                @pl.when(c + 2 < nch)
                def _():
                    issue(c + 2, 2)

                wait(1)
                accs = rowsums(buf1, (c + 1) * cw, cw, accs)

                @pl.when(c + 3 < nch)
                def _():
                    issue(c + 3, 0)

                wait(2)
                accs = rowsums(buf2, (c + 2) * cw, cw, accs)

                @pl.when(c + 4 < nch)
                def _():
                    issue(c + 4, 1)

                return accs

            accs = lax.fori_loop(0, nch // 3, triple, accs)
            for k in range(nch - nch % 3, nch):
                wait(k % 3)
                accs = rowsums(bufs[k % 3], k * cw, cw, accs)

            pltpu.make_async_copy(
                pred_hbm.at[pl.ds(row0, 8), pl.ds(nch * cw, tail)],
                tbuf, tsem).wait()
            accs = rowsums(tbuf, nch * cw, tail, accs)

            for r in range(8):
                o16[pl.ds(r * 16, 16)] = accs[r]
            pltpu.sync_copy(o16, out_hbm.at[pl.ds(row0 * 16, 128)])

    return rk(pred, target)


def _loss_body(t_ref, x_ref, o_ref, *, bc, ncols, nblk):
    j = pl.program_id(1)
    rows = x_ref.shape[0]
    cols = jax.lax.broadcasted_iota(jnp.int32, (rows, bc), 1)
    t_loc = t_ref[...] - j * bc

    def accum(s):
        @pl.when(j == 0)
        def _():
            o_ref[...] = s

        @pl.when(j > 0)
        def _():
            o_ref[...] += s

    @pl.when(j < nblk - 1)
    def _main():
        logs = jnp.log2(x_ref[...])
        accum(jnp.sum(jnp.where(cols == t_loc, 0.0, logs),
                      axis=1, keepdims=True))

    @pl.when(j == nblk - 1)
    def _last():
        nvalid = ncols - (nblk - 1) * bc
        logs = jnp.log2(x_ref[...])
        accum(jnp.sum(jnp.where((cols == t_loc) | (cols >= nvalid), 0.0, logs),
                      axis=1, keepdims=True))
        o_ref[...] = o_ref[...] * (-math.log(2.0) / ncols)


def _combine_body(scp_ref, tcs_ref, o_ref, *, ncols, rsc):
    s_sc = jnp.sum(scp_ref[...], axis=1, keepdims=True) * (-1.0 / ncols)
    o_ref[0:rsc, :] = s_sc
    o_ref[rsc:, :] = tcs_ref[...]


def kernel(pred, target):
    B, C = pred.shape
    t32 = target.astype(jnp.int32)
    sc_parts = _sc_rowsum(pred, t32, R_SC)

    n_tc = B - R_SC
    nblk = pl.cdiv(C, _BC)
    t2 = t32[R_SC:].reshape(n_tc, 1)
    tc_out = pl.pallas_call(
        functools.partial(_loss_body, bc=_BC, ncols=C, nblk=nblk),
        grid=(n_tc // R_SC, nblk),
        in_specs=[
            pl.BlockSpec((R_SC, 1), lambda i, j: (i, 0)),
            pl.BlockSpec((R_SC, _BC), lambda i, j: (i + 1, j)),
        ],
        out_specs=pl.BlockSpec((R_SC, 1), lambda i, j: (i, 0)),
        out_shape=jax.ShapeDtypeStruct((n_tc, 1), jnp.float32),
    )(t2, pred)

    out = pl.pallas_call(
        functools.partial(_combine_body, ncols=C, rsc=R_SC),
        in_specs=[
            pl.BlockSpec((R_SC, 16), lambda: (0, 0)),
            pl.BlockSpec((n_tc, 1), lambda: (0, 0)),
        ],
        out_specs=pl.BlockSpec((B, 1), lambda: (0, 0)),
        out_shape=jax.ShapeDtypeStruct((B, 1), jnp.float32),
    )(sc_parts.reshape(R_SC, 16), tc_out)
    return out[:, 0]
